# baseline (speedup 1.0000x reference)
import functools

import jax
import jax.numpy as jnp
from jax import lax
from jax.experimental import pallas as pl
from jax.experimental.pallas import tpu as pltpu
from jax.experimental.pallas import tpu_sc as plsc

H, W = 4096, 4096
N = 4194304


_BLK_ROWS = 64
_ROW = 2048


def _idx_body(cy_ref, cx_ref, idx_ref):
    cy = cy_ref[...]
    cx = cx_ref[...]
    x = cx * 2.0 - 1.0
    y = cy * 2.0 - 1.0
    ix = (x + 1.0) * 0.5 * (W - 1)
    iy = (y + 1.0) * 0.5 * (H - 1)
    ix_i = jnp.clip(jnp.round(ix), 0, W - 1).astype(jnp.int32)
    iy_i = jnp.clip(jnp.round(iy), 0, H - 1).astype(jnp.int32)
    idx_ref[...] = iy_i * W + ix_i


def _compute_indices(cy, cx):
    nrows = N // _ROW
    grid = nrows // _BLK_ROWS
    return pl.pallas_call(
        _idx_body,
        grid=(grid,),
        in_specs=[
            pl.BlockSpec((_BLK_ROWS, _ROW), lambda i: (i, 0)),
            pl.BlockSpec((_BLK_ROWS, _ROW), lambda i: (i, 0)),
        ],
        out_specs=pl.BlockSpec((_BLK_ROWS, _ROW), lambda i: (i, 0)),
        out_shape=jax.ShapeDtypeStruct((nrows, _ROW), jnp.int32),
    )(cy, cx)


_LANES = 128
_CHUNK_ROWS = 16
_NW = 32


def _make_gather():
    rows_total = N // _LANES
    rows_per_w = rows_total // _NW
    n_chunks = rows_per_w // _CHUNK_ROWS
    mesh = plsc.VectorSubcoreMesh(core_axis_name="c", subcore_axis_name="s")

    @functools.partial(
        pl.kernel,
        mesh=mesh,
        out_type=jax.ShapeDtypeStruct((rows_total, _LANES), jnp.float32),
        scratch_types=[
            pltpu.VMEM((_CHUNK_ROWS, _LANES), jnp.int32),
            pltpu.VMEM((_CHUNK_ROWS, _LANES), jnp.float32),
            pltpu.SemaphoreType.DMA,
        ],
    )
    def gather_kernel(idx_hbm, field_hbm, out_hbm, idx_v, val_v, sem):
        wid = lax.axis_index("s") * 2 + lax.axis_index("c")
        base = wid * rows_per_w

        def chunk(c, carry):
            row0 = base + c * _CHUNK_ROWS
            pltpu.sync_copy(idx_hbm.at[pl.ds(row0, _CHUNK_ROWS)], idx_v)
            copies = [
                pltpu.async_copy(field_hbm.at[idx_v.at[j]], val_v.at[j], sem)
                for j in range(_CHUNK_ROWS)
            ]
            for cp in copies:
                cp.wait()
            pltpu.sync_copy(val_v, out_hbm.at[pl.ds(row0, _CHUNK_ROWS)])
            return carry

        lax.fori_loop(0, n_chunks, chunk, 0)

    return gather_kernel


_gather = _make_gather()


@jax.jit
def kernel(coords, field):
    cy = coords[:, 0].reshape(N // _ROW, _ROW)
    cx = coords[:, 1].reshape(N // _ROW, _ROW)
    idx = _compute_indices(cy, cx).reshape(N // _LANES, _LANES)
    field_flat = field.reshape(H * W)
    vals = _gather(idx, field_flat)
    return vals.reshape(N, 1)
